# baseline (speedup 1.0000x reference)
import functools

import jax
import jax.numpy as jnp
from jax import lax
from jax.experimental import pallas as pl
from jax.experimental.pallas import tpu as pltpu
from jax.experimental.pallas import tpu_sc as plsc

_B = 4096
_F = 26
_V = 100000
_E = 32
_DENSE = 64
_OUT = _F * _E + _DENSE

_NC = 2
_NS = 16
_NW = _NC * _NS
_ROWS = _B * _F
_RPW = _ROWS // _NW
_CHUNK = 128
_NCHUNK = _RPW // _CHUNK
_LANES = 16


def _sc_gather_body(idx_hbm, table_hbm, emb_hbm, idx_v, rows_v, sem):
    wid = lax.axis_index("s") * _NC + lax.axis_index("c")

    pltpu.sync_copy(idx_hbm.at[wid], idx_v)

    def _off_body(t, _):
        r = t // (_CHUNK // _LANES)
        c = t % (_CHUNK // _LANES)
        p = r * _CHUNK + c * _LANES + lax.iota(jnp.int32, 16)
        f = lax.rem(p, jnp.int32(_F))
        cur = idx_v[r, pl.ds(c * _LANES, _LANES)]
        idx_v[r, pl.ds(c * _LANES, _LANES)] = cur + f * jnp.int32(_V)
        return _

    lax.fori_loop(0, _NCHUNK * (_CHUNK // _LANES), _off_body, 0)

    copies = [
        pltpu.async_copy(
            table_hbm.at[idx_v.at[j]],
            rows_v.at[pl.ds(j * _CHUNK, _CHUNK)],
            sem,
        )
        for j in range(_NCHUNK)
    ]
    for cp in copies:
        cp.wait()

    pltpu.sync_copy(rows_v, emb_hbm.at[pl.ds(wid * _RPW, _RPW)])


@functools.partial(jax.jit, static_argnames=())
def _sc_gather(idx_chunks, table_flat):
    mesh = plsc.VectorSubcoreMesh(core_axis_name="c", subcore_axis_name="s")
    f = pl.kernel(
        _sc_gather_body,
        mesh=mesh,
        out_type=jax.ShapeDtypeStruct((_ROWS, _E), jnp.float32),
        scratch_types=[
            pltpu.VMEM((_NCHUNK, _CHUNK), jnp.int32),
            pltpu.VMEM((_RPW, _E), jnp.float32),
            pltpu.SemaphoreType.DMA,
        ],
        compiler_params=pltpu.CompilerParams(use_tc_tiling_on_sc=False),
    )
    return f(idx_chunks, table_flat)


def _bn_body(emb_ref, dense_ref, gamma_ref, beta_ref, out_ref):
    x = jnp.concatenate([emb_ref[...], dense_ref[...]], axis=-1)
    s1 = jnp.sum(x, axis=0, keepdims=True)
    s2 = jnp.sum(x * x, axis=0, keepdims=True)
    mean = s1 * (1.0 / _B)
    var = s2 * (1.0 / _B) - mean * mean
    inv = lax.rsqrt(var + 1e-3)
    out_ref[...] = (x - mean) * (inv * gamma_ref[...]) + beta_ref[...]


def _bn(emb_flat, dense, gamma2, beta2):
    return pl.pallas_call(
        _bn_body,
        out_shape=jax.ShapeDtypeStruct((_B, _OUT), jnp.float32),
    )(emb_flat, dense, gamma2, beta2)


def kernel(indices, dense, tables, gamma, beta):
    idx_chunks = indices.reshape(_NW, _NCHUNK, _CHUNK)
    table_flat = tables.reshape(_F * _V, _E)
    emb = _sc_gather(idx_chunks, table_flat)
    emb_flat = emb.reshape(_B, _F * _E)
    gamma2 = gamma.reshape(1, _OUT)
    beta2 = beta.reshape(1, _OUT)
    return _bn(emb_flat, dense, gamma2, beta2)


# speedup vs baseline: 1.8467x; 1.8467x over previous
import functools

import jax
import jax.numpy as jnp
from jax import lax
from jax.experimental import pallas as pl
from jax.experimental.pallas import tpu as pltpu
from jax.experimental.pallas import tpu_sc as plsc

_B = 4096
_F = 26
_V = 100000
_E = 32
_DENSE = 64
_OUT = _F * _E + _DENSE

_NC = 2
_NS = 16
_NW = _NC * _NS
_ITEMS = _B * _F
_IPW = _ITEMS // _NW
_CHUNK_ITEMS = 416
_NCHUNK = _IPW // _CHUNK_ITEMS
_STREAM_IDX = 128
_NSTREAM = _CHUNK_ITEMS * _E // _STREAM_IDX
_LANES = 16


def _sc_gather_body(idx_hbm, tab_hbm, emb_hbm, idx_v, ebuf, rows, sem):
    wid = lax.axis_index("s") * _NC + lax.axis_index("c")

    pltpu.sync_copy(idx_hbm.at[wid], idx_v)

    def _off_body(t, carry):
        r = t // (128 // _LANES)
        c = t % (128 // _LANES)
        p = r * 128 + c * _LANES + lax.iota(jnp.int32, _LANES)
        f = lax.rem(p, jnp.int32(_F))
        cur = idx_v[r, pl.ds(c * _LANES, _LANES)]
        idx_v[r, pl.ds(c * _LANES, _LANES)] = cur + f * jnp.int32(_E * _V)
        return carry

    lax.fori_loop(0, 26 * (128 // _LANES), _off_body, 0)

    iota16 = lax.iota(jnp.int32, _LANES)
    e_lo = iota16 * jnp.int32(_V)
    e_hi = e_lo + jnp.int32(_LANES * _V)

    for c in range(_NCHUNK):
        def _build(g, carry):
            p0 = c * _CHUNK_ITEMS + g * _LANES
            base = idx_v[p0 // 128, pl.ds(p0 % 128, _LANES)]
            d0 = g * (_LANES * _E)
            for k in range(2 * _LANES):
                val = base[k // 2] + (e_lo if k % 2 == 0 else e_hi)
                ebuf[pl.ds(d0 + k * _LANES, _LANES)] = val
            return carry

        lax.fori_loop(0, _CHUNK_ITEMS // _LANES, _build, 0)

        def _fire(s, carry):
            pltpu.make_async_copy(
                tab_hbm.at[ebuf.at[pl.ds(s * _STREAM_IDX, _STREAM_IDX)]],
                rows.at[pl.ds(s * _STREAM_IDX, _STREAM_IDX)],
                sem,
            ).start()
            return carry

        lax.fori_loop(0, _NSTREAM, _fire, 0)

        def _drain(s, carry):
            pltpu.make_async_copy(
                tab_hbm.at[ebuf.at[pl.ds(s * _STREAM_IDX, _STREAM_IDX)]],
                rows.at[pl.ds(s * _STREAM_IDX, _STREAM_IDX)],
                sem,
            ).wait()
            return carry

        lax.fori_loop(0, _NSTREAM, _drain, 0)

        base_out = wid * _IPW * _E + c * _CHUNK_ITEMS * _E
        pltpu.sync_copy(rows, emb_hbm.at[pl.ds(base_out, _CHUNK_ITEMS * _E)])


@jax.jit
def _sc_gather(idx_chunks, tab_flat):
    mesh = plsc.VectorSubcoreMesh(core_axis_name="c", subcore_axis_name="s")
    f = pl.kernel(
        _sc_gather_body,
        mesh=mesh,
        out_type=jax.ShapeDtypeStruct((_ITEMS * _E,), jnp.float32),
        scratch_types=[
            pltpu.VMEM((_F, 128), jnp.int32),
            pltpu.VMEM((_CHUNK_ITEMS * _E,), jnp.int32),
            pltpu.VMEM((_CHUNK_ITEMS * _E,), jnp.float32),
            pltpu.SemaphoreType.DMA,
        ],
        compiler_params=pltpu.CompilerParams(use_tc_tiling_on_sc=False),
    )
    return f(idx_chunks, tab_flat)


def _bn_body(emb_ref, dense_ref, gamma_ref, beta_ref, out_ref):
    x = jnp.concatenate([emb_ref[...], dense_ref[...]], axis=-1)
    s1 = jnp.sum(x, axis=0, keepdims=True)
    s2 = jnp.sum(x * x, axis=0, keepdims=True)
    mean = s1 * (1.0 / _B)
    var = s2 * (1.0 / _B) - mean * mean
    inv = lax.rsqrt(var + 1e-3)
    out_ref[...] = (x - mean) * (inv * gamma_ref[...]) + beta_ref[...]


def _bn(emb_flat, dense, gamma2, beta2):
    return pl.pallas_call(
        _bn_body,
        out_shape=jax.ShapeDtypeStruct((_B, _OUT), jnp.float32),
    )(emb_flat, dense, gamma2, beta2)


def kernel(indices, dense, tables, gamma, beta):
    idx_chunks = indices.reshape(_NW, _F, 128)
    tab_flat = jnp.transpose(tables, (0, 2, 1)).reshape(-1)
    emb1d = _sc_gather(idx_chunks, tab_flat)
    emb_flat = emb1d.reshape(_B, _F * _E)
    gamma2 = gamma.reshape(1, _OUT)
    beta2 = beta.reshape(1, _OUT)
    return _bn(emb_flat, dense, gamma2, beta2)
